# baseline (speedup 1.0000x reference)
import functools

import jax
import jax.numpy as jnp
from jax import lax
from jax.experimental import pallas as pl
from jax.experimental.pallas import tpu as pltpu
from jax.experimental.pallas import tpu_sc as plsc

D = 128
NC = 2
NHALF = 2
NSLICE = 8
F = 16
CE = 1024
GPC = CE // 128
NGRP = CE // 16
NEG_INF = float("-inf")


def _make_segmax(npad, e_pad):
  e_per_core = e_pad // NC
  nchunk = e_per_core // CE
  rows_per_core = e_per_core // 128
  nh = npad // NHALF

  mesh = plsc.VectorSubcoreMesh(
      core_axis_name="c", subcore_axis_name="s", num_cores=NC,
      num_subcores=NHALF * NSLICE)

  @functools.partial(
      pl.kernel,
      compiler_params=pltpu.CompilerParams(use_tc_tiling_on_sc=False),
      out_type=jax.ShapeDtypeStruct((NC, npad, D), jnp.float32),
      mesh=mesh,
      scratch_types=[
          pltpu.VMEM((2, GPC, 2, 128), jnp.int32),
          pltpu.VMEM((2, CE, F), jnp.float32),
          pltpu.VMEM(((nh + 1) * F,), jnp.float32),
          pltpu.SemaphoreType.DMA((2,)),
          pltpu.SemaphoreType.DMA((2,)),
      ],
  )
  def segmax(xs_hbm, ed_hbm, out_hbm, ebuf, stripebuf, acc, sg, ss):
    c = lax.axis_index("c")
    s = lax.axis_index("s")
    h = s // NSLICE
    f = s % NSLICE
    hlo = h * nh

    neg = jnp.full((F,), NEG_INF, jnp.float32)

    @pl.loop(0, nh + 1)
    def _init_acc(i):
      acc[pl.ds(i * F, F)] = neg

    rbase = c * rows_per_core

    def stage_start(ch, buf):
      rb = rbase + ch * GPC
      return pltpu.make_async_copy(
          ed_hbm.at[pl.ds(rb, GPC)], ebuf.at[buf], ss.at[buf])

    def fire_gathers(buf):
      for k in range(GPC):
        pltpu.async_copy(
            xs_hbm.at[f].at[ebuf.at[buf, k, 0]],
            stripebuf.at[buf, pl.ds(k * 128, 128)],
            sg.at[buf])

    def drain_gathers(buf):
      for _ in range(GPC):
        pltpu.make_async_copy(
            xs_hbm.at[0].at[ebuf.at[buf, 0, 0]],
            stripebuf.at[buf, pl.ds(0, 128)],
            sg.at[buf]).wait()

    stage_start(0, 0).start()
    stage_start(0, 0).wait()
    fire_gathers(0)

    @pl.loop(0, nchunk)
    def _chunk(ch):
      buf = lax.rem(ch, 2)
      nxt = 1 - buf
      nxt_ch = jnp.minimum(ch + 1, nchunk - 1)
      st = stage_start(nxt_ch, nxt)
      st.start()
      drain_gathers(buf)
      st.wait()
      fire_gathers(nxt)

      @pl.loop(0, NGRP, unroll=2)
      def _grp(g):
        dvec = ebuf[buf, g // 8, 1, pl.ds((g % 8) * 16, 16)]
        dloc = dvec - hlo
        inr = (dloc >= 0) & (dloc < nh)
        rows_w = jnp.where(inr, dloc, nh) * F
        rows = [rows_w[lane] for lane in range(16)]
        a = acc[pl.ds(rows[0], F)]
        nxt_load = acc[pl.ds(rows[1], F)]
        for lane in range(16):
          m = jnp.maximum(a, stripebuf[buf, g * 16 + lane, :])
          acc[pl.ds(rows[lane], F)] = m
          new_load = acc[pl.ds(rows[min(lane + 2, 15)], F)]
          nl = min(lane + 1, 15)
          a = jnp.where(rows[nl] == rows[lane], m, nxt_load)
          nxt_load = new_load

    drain_gathers(lax.rem(nchunk, 2))
    for pblk in range(nh // CE):

      @pl.loop(0, CE)
      def _rp(i):
        stripebuf[0, i, :] = acc[pl.ds((pblk * CE + i) * F, F)]

      pltpu.sync_copy(
          stripebuf.at[0],
          out_hbm.at[c, pl.ds(hlo + pblk * CE, CE), pl.ds(f * F, F)])

  return segmax


def _sage_tc_body(agg_ref, x_ref, wl_ref, wr_ref, b_ref, o_ref, os_ref):
  a = jnp.max(agg_ref[...], axis=0)
  a = jnp.where(a == NEG_INF, 0.0, a)
  out = jax.nn.relu(
      jnp.dot(a, wl_ref[...], preferred_element_type=jnp.float32)
      + jnp.dot(x_ref[...], wr_ref[...], preferred_element_type=jnp.float32)
      + b_ref[...])
  o_ref[...] = out
  for f in range(NSLICE):
    os_ref[f] = out[:, f * F:(f + 1) * F]


def _sage_tc(aggp, xin, wl, wr, b, bm=256):
  npad = xin.shape[0]
  grid = (npad // bm,)
  return pl.pallas_call(
      _sage_tc_body,
      grid=grid,
      in_specs=[
          pl.BlockSpec((NC, bm, D), lambda i: (0, i, 0)),
          pl.BlockSpec((bm, D), lambda i: (i, 0)),
          pl.BlockSpec((D, D), lambda i: (0, 0)),
          pl.BlockSpec((D, D), lambda i: (0, 0)),
          pl.BlockSpec((1, D), lambda i: (0, 0)),
      ],
      out_specs=[
          pl.BlockSpec((bm, D), lambda i: (i, 0)),
          pl.BlockSpec((NSLICE, bm, F), lambda i: (0, i, 0)),
      ],
      out_shape=[
          jax.ShapeDtypeStruct((npad, D), jnp.float32),
          jax.ShapeDtypeStruct((NSLICE, npad, F), jnp.float32),
      ],
  )(aggp, xin, wl, wr, b.reshape(1, D))


def _head_body(agg_ref, h_ref, wlh_ref, wrh_ref, bh_ref, p_ref, bias_ref,
               o_ref):
  a = jnp.max(agg_ref[...], axis=0)
  a = jnp.where(a == NEG_INF, 0.0, a)
  hh = jax.nn.relu(
      jnp.dot(a, wlh_ref[...], preferred_element_type=jnp.float32)
      + jnp.dot(h_ref[...], wrh_ref[...], preferred_element_type=jnp.float32)
      + bh_ref[...])
  o_ref[...] = (
      jnp.dot(hh, p_ref[...], preferred_element_type=jnp.float32)
      + bias_ref[...])


def _heads_tc(aggp, hx, wlh, wrh, bh, p, bias, bm=256):
  npad = hx.shape[0]
  hw = wlh.shape[1]
  grid = (npad // bm,)
  return pl.pallas_call(
      _head_body,
      grid=grid,
      in_specs=[
          pl.BlockSpec((NC, bm, D), lambda i: (0, i, 0)),
          pl.BlockSpec((bm, D), lambda i: (i, 0)),
          pl.BlockSpec((D, hw), lambda i: (0, 0)),
          pl.BlockSpec((D, hw), lambda i: (0, 0)),
          pl.BlockSpec((1, hw), lambda i: (0, 0)),
          pl.BlockSpec((hw, D), lambda i: (0, 0)),
          pl.BlockSpec((1, D), lambda i: (0, 0)),
      ],
      out_specs=pl.BlockSpec((bm, D), lambda i: (i, 0)),
      out_shape=jax.ShapeDtypeStruct((npad, D), jnp.float32),
  )(aggp, hx, wlh, wrh, bh, p, bias)


def kernel(x, edge_index, Wl1, Wr1, b1, Wl2, Wr2, b2,
           Wlr, Wrr, br, Wlinr, blinr,
           Wlm, Wrm, bm, Wlinm, blinm,
           Wlj, Wrj, bj, Wlinj, blinj):
  n = x.shape[0]
  e = edge_index.shape[1]
  npad = ((n + 255) // 256) * 256
  estep = NC * CE
  e_pad = ((e + estep - 1) // estep) * estep

  xp = jnp.pad(x, ((0, npad - n), (0, 0)))
  src = jnp.pad(edge_index[0], (0, e_pad - e))
  dst = jnp.pad(edge_index[1], (0, e_pad - e), constant_values=npad)
  ed = jnp.stack([src.reshape(-1, 128), dst.reshape(-1, 128)], axis=1)

  segmax = _make_segmax(npad, e_pad)

  xs1 = xp.reshape(npad, NSLICE, F).transpose(1, 0, 2)
  aggp1 = segmax(xs1, ed)
  h1, h1s = _sage_tc(aggp1, xp, Wl1, Wr1, b1)
  aggp2 = segmax(h1s, ed)
  h2, h2s = _sage_tc(aggp2, h1, Wl2, Wr2, b2)
  aggp3 = segmax(h2s, ed)

  wlh = jnp.concatenate([Wlr, Wlm, Wlj], axis=1)
  wrh = jnp.concatenate([Wrr, Wrm, Wrj], axis=1)
  bh = jnp.concatenate([br, bm, bj]).reshape(1, -1)
  h2w = Wlinr.shape[0]
  p = jnp.zeros((3 * h2w, D), jnp.float32)
  p = p.at[0:h2w, 0:1].set(Wlinr)
  p = p.at[h2w:2 * h2w, 1:2].set(Wlinm)
  p = p.at[2 * h2w:3 * h2w, 2:4].set(Wlinj)
  bias = jnp.zeros((1, D), jnp.float32)
  bias = bias.at[0, 0:1].set(blinr)
  bias = bias.at[0, 1:2].set(blinm)
  bias = bias.at[0, 2:4].set(blinj)

  out = _heads_tc(aggp3, h2, wlh, wrh, bh, p, bias)
  rt = out[:n, 0]
  md = out[:n, 1]
  jr = out[:n, 2:4].reshape(-1)
  return (rt, md, jr)


# speedup vs baseline: 1.0124x; 1.0124x over previous
import functools

import jax
import jax.numpy as jnp
from jax import lax
from jax.experimental import pallas as pl
from jax.experimental.pallas import tpu as pltpu
from jax.experimental.pallas import tpu_sc as plsc

D = 128
NC = 2
NHALF = 2
NSLICE = 8
F = 16
CE = 1024
GPC = CE // 128
NGRP = CE // 16
NEG_INF = float("-inf")


def _make_segmax(npad, e_pad):
  e_per_core = e_pad // NC
  nchunk = e_per_core // CE
  rows_per_core = e_per_core // 128
  nh = npad // NHALF

  mesh = plsc.VectorSubcoreMesh(
      core_axis_name="c", subcore_axis_name="s", num_cores=NC,
      num_subcores=NHALF * NSLICE)

  @functools.partial(
      pl.kernel,
      compiler_params=pltpu.CompilerParams(use_tc_tiling_on_sc=False),
      out_type=jax.ShapeDtypeStruct((NC, npad, D), jnp.float32),
      mesh=mesh,
      scratch_types=[
          pltpu.VMEM((2, GPC, 2, 128), jnp.int32),
          pltpu.VMEM((2, CE, F), jnp.float32),
          pltpu.VMEM((nh + 1, F), jnp.float32),
          pltpu.SemaphoreType.DMA((2,)),
          pltpu.SemaphoreType.DMA((2,)),
      ],
  )
  def segmax(xs_hbm, ed_hbm, out_hbm, ebuf, stripebuf, acc, sg, ss):
    c = lax.axis_index("c")
    s = lax.axis_index("s")
    h = s // NSLICE
    f = s % NSLICE
    hlo = h * nh

    neg = jnp.full((F,), NEG_INF, jnp.float32)

    @pl.loop(0, nh + 1)
    def _init_acc(i):
      acc[i, :] = neg

    rbase = c * rows_per_core

    def stage_start(ch, buf):
      rb = rbase + ch * GPC
      return pltpu.make_async_copy(
          ed_hbm.at[pl.ds(rb, GPC)], ebuf.at[buf], ss.at[buf])

    def fire_gathers(buf):
      for k in range(GPC):
        pltpu.async_copy(
            xs_hbm.at[f].at[ebuf.at[buf, k, 0]],
            stripebuf.at[buf, pl.ds(k * 128, 128)],
            sg.at[buf])

    def drain_gathers(buf):
      for _ in range(GPC):
        pltpu.make_async_copy(
            xs_hbm.at[0].at[ebuf.at[buf, 0, 0]],
            stripebuf.at[buf, pl.ds(0, 128)],
            sg.at[buf]).wait()

    stage_start(0, 0).start()
    stage_start(0, 0).wait()
    fire_gathers(0)

    @pl.loop(0, nchunk)
    def _chunk(ch):
      buf = lax.rem(ch, 2)
      nxt = 1 - buf
      nxt_ch = jnp.minimum(ch + 1, nchunk - 1)
      st = stage_start(nxt_ch, nxt)
      st.start()
      drain_gathers(buf)
      st.wait()
      fire_gathers(nxt)

      @pl.loop(0, NGRP, unroll=4)
      def _grp(g):
        dvec = ebuf[buf, g // 8, 1, pl.ds((g % 8) * 16, 16)]
        dloc = dvec - hlo
        inr = (dloc >= 0) & (dloc < nh)
        drows = jnp.where(inr, dloc, nh)
        rows = [drows[lane] for lane in range(16)]
        a = acc[rows[0], :]
        nxt_load = acc[rows[1], :]
        for lane in range(16):
          m = jnp.maximum(a, stripebuf[buf, g * 16 + lane, :])
          acc[rows[lane], :] = m
          new_load = acc[rows[min(lane + 2, 15)], :]
          nl = min(lane + 1, 15)
          a = jnp.where(rows[nl] == rows[lane], m, nxt_load)
          nxt_load = new_load

    drain_gathers(lax.rem(nchunk, 2))
    pltpu.sync_copy(acc.at[pl.ds(0, nh)],
                    out_hbm.at[c, pl.ds(hlo, nh), pl.ds(f * F, F)])

  return segmax


def _sage_tc_body(agg_ref, x_ref, wl_ref, wr_ref, b_ref, o_ref, os_ref):
  a = jnp.max(agg_ref[...], axis=0)
  a = jnp.where(a == NEG_INF, 0.0, a)
  out = jax.nn.relu(
      jnp.dot(a, wl_ref[...], preferred_element_type=jnp.float32)
      + jnp.dot(x_ref[...], wr_ref[...], preferred_element_type=jnp.float32)
      + b_ref[...])
  o_ref[...] = out
  for f in range(NSLICE):
    os_ref[f] = out[:, f * F:(f + 1) * F]


def _sage_tc(aggp, xin, wl, wr, b, bm=256):
  npad = xin.shape[0]
  grid = (npad // bm,)
  return pl.pallas_call(
      _sage_tc_body,
      grid=grid,
      in_specs=[
          pl.BlockSpec((NC, bm, D), lambda i: (0, i, 0)),
          pl.BlockSpec((bm, D), lambda i: (i, 0)),
          pl.BlockSpec((D, D), lambda i: (0, 0)),
          pl.BlockSpec((D, D), lambda i: (0, 0)),
          pl.BlockSpec((1, D), lambda i: (0, 0)),
      ],
      out_specs=[
          pl.BlockSpec((bm, D), lambda i: (i, 0)),
          pl.BlockSpec((NSLICE, bm, F), lambda i: (0, i, 0)),
      ],
      out_shape=[
          jax.ShapeDtypeStruct((npad, D), jnp.float32),
          jax.ShapeDtypeStruct((NSLICE, npad, F), jnp.float32),
      ],
  )(aggp, xin, wl, wr, b.reshape(1, D))


def _head_body(agg_ref, h_ref, wlh_ref, wrh_ref, bh_ref, p_ref, bias_ref,
               o_ref):
  a = jnp.max(agg_ref[...], axis=0)
  a = jnp.where(a == NEG_INF, 0.0, a)
  hh = jax.nn.relu(
      jnp.dot(a, wlh_ref[...], preferred_element_type=jnp.float32)
      + jnp.dot(h_ref[...], wrh_ref[...], preferred_element_type=jnp.float32)
      + bh_ref[...])
  o_ref[...] = (
      jnp.dot(hh, p_ref[...], preferred_element_type=jnp.float32)
      + bias_ref[...])


def _heads_tc(aggp, hx, wlh, wrh, bh, p, bias, bm=256):
  npad = hx.shape[0]
  hw = wlh.shape[1]
  grid = (npad // bm,)
  return pl.pallas_call(
      _head_body,
      grid=grid,
      in_specs=[
          pl.BlockSpec((NC, bm, D), lambda i: (0, i, 0)),
          pl.BlockSpec((bm, D), lambda i: (i, 0)),
          pl.BlockSpec((D, hw), lambda i: (0, 0)),
          pl.BlockSpec((D, hw), lambda i: (0, 0)),
          pl.BlockSpec((1, hw), lambda i: (0, 0)),
          pl.BlockSpec((hw, D), lambda i: (0, 0)),
          pl.BlockSpec((1, D), lambda i: (0, 0)),
      ],
      out_specs=pl.BlockSpec((bm, D), lambda i: (i, 0)),
      out_shape=jax.ShapeDtypeStruct((npad, D), jnp.float32),
  )(aggp, hx, wlh, wrh, bh, p, bias)


def kernel(x, edge_index, Wl1, Wr1, b1, Wl2, Wr2, b2,
           Wlr, Wrr, br, Wlinr, blinr,
           Wlm, Wrm, bm, Wlinm, blinm,
           Wlj, Wrj, bj, Wlinj, blinj):
  n = x.shape[0]
  e = edge_index.shape[1]
  npad = ((n + 255) // 256) * 256
  estep = NC * CE
  e_pad = ((e + estep - 1) // estep) * estep

  xp = jnp.pad(x, ((0, npad - n), (0, 0)))
  src = jnp.pad(edge_index[0], (0, e_pad - e))
  dst = jnp.pad(edge_index[1], (0, e_pad - e), constant_values=npad)
  ed = jnp.stack([src.reshape(-1, 128), dst.reshape(-1, 128)], axis=1)

  segmax = _make_segmax(npad, e_pad)

  xs1 = xp.reshape(npad, NSLICE, F).transpose(1, 0, 2)
  aggp1 = segmax(xs1, ed)
  h1, h1s = _sage_tc(aggp1, xp, Wl1, Wr1, b1)
  aggp2 = segmax(h1s, ed)
  h2, h2s = _sage_tc(aggp2, h1, Wl2, Wr2, b2)
  aggp3 = segmax(h2s, ed)

  wlh = jnp.concatenate([Wlr, Wlm, Wlj], axis=1)
  wrh = jnp.concatenate([Wrr, Wrm, Wrj], axis=1)
  bh = jnp.concatenate([br, bm, bj]).reshape(1, -1)
  h2w = Wlinr.shape[0]
  p = jnp.zeros((3 * h2w, D), jnp.float32)
  p = p.at[0:h2w, 0:1].set(Wlinr)
  p = p.at[h2w:2 * h2w, 1:2].set(Wlinm)
  p = p.at[2 * h2w:3 * h2w, 2:4].set(Wlinj)
  bias = jnp.zeros((1, D), jnp.float32)
  bias = bias.at[0, 0:1].set(blinr)
  bias = bias.at[0, 1:2].set(blinm)
  bias = bias.at[0, 2:4].set(blinj)

  out = _heads_tc(aggp3, h2, wlh, wrh, bh, p, bias)
  rt = out[:n, 0]
  md = out[:n, 1]
  jr = out[:n, 2:4].reshape(-1)
  return (rt, md, jr)


# speedup vs baseline: 1.0182x; 1.0057x over previous
import functools

import jax
import jax.numpy as jnp
from jax import lax
from jax.experimental import pallas as pl
from jax.experimental.pallas import tpu as pltpu
from jax.experimental.pallas import tpu_sc as plsc

D = 128
NC = 2
NHALF = 2
NSLICE = 8
F = 16
CE = 1024
GPC = CE // 128
NGRP = CE // 16
NEG_INF = float("-inf")


def _make_segmax(npad, e_pad):
  e_per_core = e_pad // NC
  nchunk = e_per_core // CE
  rows_per_core = e_per_core // 128
  nh = npad // NHALF

  mesh = plsc.VectorSubcoreMesh(
      core_axis_name="c", subcore_axis_name="s", num_cores=NC,
      num_subcores=NHALF * NSLICE)

  @functools.partial(
      pl.kernel,
      compiler_params=pltpu.CompilerParams(use_tc_tiling_on_sc=False),
      out_type=jax.ShapeDtypeStruct((NC, npad, D), jnp.float32),
      mesh=mesh,
      scratch_types=[
          pltpu.VMEM((2, GPC, 2, 128), jnp.int32),
          pltpu.VMEM((2, CE, F), jnp.float32),
          pltpu.VMEM((nh + 1, F), jnp.float32),
          pltpu.SemaphoreType.DMA((2,)),
          pltpu.SemaphoreType.DMA((2,)),
      ],
  )
  def segmax(xs_hbm, ed_hbm, out_hbm, ebuf, stripebuf, acc, sg, ss):
    c = lax.axis_index("c")
    s = lax.axis_index("s")
    h = s // NSLICE
    f = s % NSLICE
    hlo = h * nh

    neg = jnp.full((F,), NEG_INF, jnp.float32)

    @pl.loop(0, nh + 1)
    def _init_acc(i):
      acc[i, :] = neg

    rbase = c * rows_per_core

    def stage_start(ch, buf):
      rb = rbase + ch * GPC
      return pltpu.make_async_copy(
          ed_hbm.at[pl.ds(rb, GPC)], ebuf.at[buf], ss.at[buf])

    def fire_gathers(buf):
      for k in range(GPC):
        pltpu.async_copy(
            xs_hbm.at[f].at[ebuf.at[buf, k, 0]],
            stripebuf.at[buf, pl.ds(k * 128, 128)],
            sg.at[buf])

    def drain_gathers(buf):
      for _ in range(GPC):
        pltpu.make_async_copy(
            xs_hbm.at[0].at[ebuf.at[buf, 0, 0]],
            stripebuf.at[buf, pl.ds(0, 128)],
            sg.at[buf]).wait()

    stage_start(0, 0).start()
    stage_start(0, 0).wait()
    fire_gathers(0)

    @pl.loop(0, nchunk)
    def _chunk(ch):
      buf = lax.rem(ch, 2)
      nxt = 1 - buf
      nxt_ch = jnp.minimum(ch + 1, nchunk - 1)
      st = stage_start(nxt_ch, nxt)
      st.start()
      drain_gathers(buf)
      st.wait()
      fire_gathers(nxt)

      @pl.loop(0, NGRP, unroll=2)
      def _grp(g):
        dvec = ebuf[buf, g // 8, 1, pl.ds((g % 8) * 16, 16)]
        dloc = dvec - hlo
        inr = (dloc >= 0) & (dloc < nh)
        drows = jnp.where(inr, dloc, nh)
        rows = [drows[lane] for lane in range(16)]
        a = acc[rows[0], :]
        nxt_load = acc[rows[1], :]
        for lane in range(16):
          m = jnp.maximum(a, stripebuf[buf, g * 16 + lane, :])
          acc[rows[lane], :] = m
          new_load = acc[rows[min(lane + 2, 15)], :]
          nl = min(lane + 1, 15)
          a = jnp.where(rows[nl] == rows[lane], m, nxt_load)
          nxt_load = new_load

    drain_gathers(lax.rem(nchunk, 2))
    pltpu.sync_copy(acc.at[pl.ds(0, nh)],
                    out_hbm.at[c, pl.ds(hlo, nh), pl.ds(f * F, F)])

  return segmax


def _sage_tc_body(agg_ref, x_ref, wl_ref, wr_ref, b_ref, o_ref, os_ref):
  a = jnp.max(agg_ref[...], axis=0)
  a = jnp.where(a == NEG_INF, 0.0, a)
  out = jax.nn.relu(
      jnp.dot(a, wl_ref[...], preferred_element_type=jnp.float32)
      + jnp.dot(x_ref[...], wr_ref[...], preferred_element_type=jnp.float32)
      + b_ref[...])
  o_ref[...] = out
  for f in range(NSLICE):
    os_ref[f] = out[:, f * F:(f + 1) * F]


def _sage_tc(aggp, xin, wl, wr, b, bm=256):
  npad = xin.shape[0]
  grid = (npad // bm,)
  return pl.pallas_call(
      _sage_tc_body,
      grid=grid,
      in_specs=[
          pl.BlockSpec((NC, bm, D), lambda i: (0, i, 0)),
          pl.BlockSpec((bm, D), lambda i: (i, 0)),
          pl.BlockSpec((D, D), lambda i: (0, 0)),
          pl.BlockSpec((D, D), lambda i: (0, 0)),
          pl.BlockSpec((1, D), lambda i: (0, 0)),
      ],
      out_specs=[
          pl.BlockSpec((bm, D), lambda i: (i, 0)),
          pl.BlockSpec((NSLICE, bm, F), lambda i: (0, i, 0)),
      ],
      out_shape=[
          jax.ShapeDtypeStruct((npad, D), jnp.float32),
          jax.ShapeDtypeStruct((NSLICE, npad, F), jnp.float32),
      ],
  )(aggp, xin, wl, wr, b.reshape(1, D))


def _head_body(agg_ref, h_ref, wlh_ref, wrh_ref, bh_ref, p_ref, bias_ref,
               o_ref):
  a = jnp.max(agg_ref[...], axis=0)
  a = jnp.where(a == NEG_INF, 0.0, a)
  hh = jax.nn.relu(
      jnp.dot(a, wlh_ref[...], preferred_element_type=jnp.float32)
      + jnp.dot(h_ref[...], wrh_ref[...], preferred_element_type=jnp.float32)
      + bh_ref[...])
  o_ref[...] = (
      jnp.dot(hh, p_ref[...], preferred_element_type=jnp.float32)
      + bias_ref[...])


def _heads_tc(aggp, hx, wlh, wrh, bh, p, bias, bm=256):
  npad = hx.shape[0]
  hw = wlh.shape[1]
  grid = (npad // bm,)
  return pl.pallas_call(
      _head_body,
      grid=grid,
      in_specs=[
          pl.BlockSpec((NC, bm, D), lambda i: (0, i, 0)),
          pl.BlockSpec((bm, D), lambda i: (i, 0)),
          pl.BlockSpec((D, hw), lambda i: (0, 0)),
          pl.BlockSpec((D, hw), lambda i: (0, 0)),
          pl.BlockSpec((1, hw), lambda i: (0, 0)),
          pl.BlockSpec((hw, D), lambda i: (0, 0)),
          pl.BlockSpec((1, D), lambda i: (0, 0)),
      ],
      out_specs=pl.BlockSpec((bm, D), lambda i: (i, 0)),
      out_shape=jax.ShapeDtypeStruct((npad, D), jnp.float32),
  )(aggp, hx, wlh, wrh, bh, p, bias)


def kernel(x, edge_index, Wl1, Wr1, b1, Wl2, Wr2, b2,
           Wlr, Wrr, br, Wlinr, blinr,
           Wlm, Wrm, bm, Wlinm, blinm,
           Wlj, Wrj, bj, Wlinj, blinj):
  n = x.shape[0]
  e = edge_index.shape[1]
  npad = ((n + 255) // 256) * 256
  estep = NC * CE
  e_pad = ((e + estep - 1) // estep) * estep

  xp = jnp.pad(x, ((0, npad - n), (0, 0)))
  src = jnp.pad(edge_index[0], (0, e_pad - e))
  dst = jnp.pad(edge_index[1], (0, e_pad - e), constant_values=npad)
  ed = jnp.stack([src.reshape(-1, 128), dst.reshape(-1, 128)], axis=1)

  segmax = _make_segmax(npad, e_pad)

  xs1 = xp.reshape(npad, NSLICE, F).transpose(1, 0, 2)
  aggp1 = segmax(xs1, ed)
  h1, h1s = _sage_tc(aggp1, xp, Wl1, Wr1, b1)
  aggp2 = segmax(h1s, ed)
  h2, h2s = _sage_tc(aggp2, h1, Wl2, Wr2, b2)
  aggp3 = segmax(h2s, ed)

  wlh = jnp.concatenate([Wlr, Wlm, Wlj], axis=1)
  wrh = jnp.concatenate([Wrr, Wrm, Wrj], axis=1)
  bh = jnp.concatenate([br, bm, bj]).reshape(1, -1)
  h2w = Wlinr.shape[0]
  p = jnp.zeros((3 * h2w, D), jnp.float32)
  p = p.at[0:h2w, 0:1].set(Wlinr)
  p = p.at[h2w:2 * h2w, 1:2].set(Wlinm)
  p = p.at[2 * h2w:3 * h2w, 2:4].set(Wlinj)
  bias = jnp.zeros((1, D), jnp.float32)
  bias = bias.at[0, 0:1].set(blinr)
  bias = bias.at[0, 1:2].set(blinm)
  bias = bias.at[0, 2:4].set(blinj)

  out = _heads_tc(aggp3, h2, wlh, wrh, bh, p, bias)
  rt = out[:n, 0]
  md = out[:n, 1]
  jr = out[:n, 2:4].reshape(-1)
  return (rt, md, jr)
